# baseline (speedup 1.0000x reference)
import functools

import jax
import jax.numpy as jnp
from jax import lax
from jax.experimental import pallas as pl
from jax.experimental.pallas import tpu as pltpu
from jax.experimental.pallas import tpu_sc as plsc

B = 4096
L = 20
V = 64
D = 4
NUM_TABLES = 3
NW = 32
ROWS_PER_W = B // NW
GROUPS = ROWS_PER_W // 16

BL = B * L
OFF_DN = 0
OFF_TU = OFF_DN + B * 2
OFF_TC = OFF_TU + V * D
OFF_TK = OFF_TC + V * D
OFF_PB = OFF_TK + V * D
TOTAL = OFF_PB + 16 * 16


def _f32(x):
    return plsc.bitcast(x, jnp.float32)


def _sc_kernel_body(ui, ci, ki, blob, out,
                    ui_v, ci_v, ki_v, dn_v, tu_v, tc_v, tk_v, pb_v,
                    su_v, sc_v, sk_v, out_v, sem):
    wid = lax.axis_index("s") * 2 + lax.axis_index("c")
    base = wid * ROWS_PER_W

    cps = [
        pltpu.async_copy(ui.at[pl.ds(base * L, ROWS_PER_W * L)], ui_v, sem),
        pltpu.async_copy(ci.at[pl.ds(base * L, ROWS_PER_W * L)], ci_v, sem),
        pltpu.async_copy(ki.at[pl.ds(base * L, ROWS_PER_W * L)], ki_v, sem),
        pltpu.async_copy(blob.at[pl.ds(OFF_DN + base * 2, ROWS_PER_W * 2)], dn_v, sem),
        pltpu.async_copy(blob.at[pl.ds(OFF_TU, V * D)], tu_v, sem),
        pltpu.async_copy(blob.at[pl.ds(OFF_TC, V * D)], tc_v, sem),
        pltpu.async_copy(blob.at[pl.ds(OFF_TK, V * D)], tk_v, sem),
        pltpu.async_copy(blob.at[pl.ds(OFF_PB, 16 * 16)], pb_v, sem),
    ]
    for cp in cps:
        cp.wait()

    iota = lax.iota(jnp.int32, 16)
    iota4 = iota * 4
    iota2 = iota * 2
    iotaL = iota * L

    for t, (tv, sv) in enumerate(((tu_v, su_v), (tc_v, sc_v), (tk_v, sk_v))):
        for c in range(V // 16):
            acc = jnp.zeros((16,), jnp.float32)
            for d in range(D):
                col = _f32(plsc.load_gather(tv, [iota4 + (c * 64 + d)]))
                acc = acc + col * _f32(pb_v[pl.ds((t * D + d) * 16, 16)])
            sv[pl.ds(c * 16, 16)] = acc

    w12 = _f32(pb_v[pl.ds(NUM_TABLES * D * 16, 16)])
    w13 = _f32(pb_v[pl.ds((NUM_TABLES * D + 1) * 16, 16)])
    bias = _f32(pb_v[pl.ds((NUM_TABLES * D + 2) * 16, 16)])

    for g in range(GROUPS):
        off = g * (16 * L)
        d0 = _f32(plsc.load_gather(dn_v, [iota2 + g * 32]))
        d1 = _f32(plsc.load_gather(dn_v, [iota2 + g * 32 + 1]))
        acc = bias + d0 * w12 + d1 * w13
        for iv, sv in ((ui_v, su_v), (ci_v, sc_v), (ki_v, sk_v)):
            for l in range(L):
                vi = plsc.load_gather(iv, [iotaL + (off + l)])
                acc = acc + plsc.load_gather(sv, [vi])
        out_v[pl.ds(g * 16, 16)] = acc

    pltpu.sync_copy(out_v, out.at[pl.ds(base, ROWS_PER_W)])


def _bits(x):
    return lax.bitcast_convert_type(x.astype(jnp.float32), jnp.int32).reshape(-1)


@jax.jit
def kernel(user_indices, context_indices, candidate_indices, dense_features,
           table_user, table_context, table_candidate, W, b):
    params = jnp.concatenate([W.astype(jnp.float32).reshape(-1),
                              b.astype(jnp.float32).reshape(-1),
                              jnp.zeros((1,), jnp.float32)])
    pb = jnp.broadcast_to(params[:, None], (16, 16))
    blob = jnp.concatenate([
        _bits(dense_features),
        _bits(table_user),
        _bits(table_context),
        _bits(table_candidate),
        _bits(pb),
    ])
    ui = user_indices.astype(jnp.int32).reshape(-1)
    ci = context_indices.astype(jnp.int32).reshape(-1)
    ki = candidate_indices.astype(jnp.int32).reshape(-1)

    run = functools.partial(
        pl.kernel,
        mesh=plsc.VectorSubcoreMesh(core_axis_name="c", subcore_axis_name="s"),
        out_type=jax.ShapeDtypeStruct((B,), jnp.float32),
        compiler_params=pltpu.CompilerParams(needs_layout_passes=False),
        scratch_types=[
            pltpu.VMEM((ROWS_PER_W * L,), jnp.int32),
            pltpu.VMEM((ROWS_PER_W * L,), jnp.int32),
            pltpu.VMEM((ROWS_PER_W * L,), jnp.int32),
            pltpu.VMEM((ROWS_PER_W * 2,), jnp.int32),
            pltpu.VMEM((V * D,), jnp.int32),
            pltpu.VMEM((V * D,), jnp.int32),
            pltpu.VMEM((V * D,), jnp.int32),
            pltpu.VMEM((16 * 16,), jnp.int32),
            pltpu.VMEM((V,), jnp.float32),
            pltpu.VMEM((V,), jnp.float32),
            pltpu.VMEM((V,), jnp.float32),
            pltpu.VMEM((ROWS_PER_W,), jnp.float32),
            pltpu.SemaphoreType.DMA,
        ],
    )(_sc_kernel_body)
    return run(ui, ci, ki, blob)


# speedup vs baseline: 1.0508x; 1.0508x over previous
import functools

import jax
import jax.numpy as jnp
from jax import lax
from jax.experimental import pallas as pl
from jax.experimental.pallas import tpu as pltpu
from jax.experimental.pallas import tpu_sc as plsc

B = 4096
L = 20
V = 64
D = 4
NUM_TABLES = 3
NW = 32
ROWS_PER_W = B // NW
GROUPS = ROWS_PER_W // 16

BL = B * L
OFF_UI = 0
OFF_CI = BL
OFF_KI = 2 * BL
OFF_DN = 3 * BL
OFF_TU = OFF_DN + B * 2
OFF_TC = OFF_TU + V * D
OFF_TK = OFF_TC + V * D
OFF_PB = OFF_TK + V * D
TOTAL = OFF_PB + 16 * 16


def _f32(x):
    return plsc.bitcast(x, jnp.float32)


def _sc_kernel_body(blob, out,
                    ui_v, ci_v, ki_v, dn_v, tu_v, tc_v, tk_v, pb_v,
                    su_v, sc_v, sk_v, out_v, sem):
    wid = lax.axis_index("s") * 2 + lax.axis_index("c")
    base = wid * ROWS_PER_W

    cps = [
        pltpu.async_copy(blob.at[pl.ds(OFF_UI + base * L, ROWS_PER_W * L)], ui_v, sem),
        pltpu.async_copy(blob.at[pl.ds(OFF_CI + base * L, ROWS_PER_W * L)], ci_v, sem),
        pltpu.async_copy(blob.at[pl.ds(OFF_KI + base * L, ROWS_PER_W * L)], ki_v, sem),
        pltpu.async_copy(blob.at[pl.ds(OFF_DN + base * 2, ROWS_PER_W * 2)], dn_v, sem),
        pltpu.async_copy(blob.at[pl.ds(OFF_TU, V * D)], tu_v, sem),
        pltpu.async_copy(blob.at[pl.ds(OFF_TC, V * D)], tc_v, sem),
        pltpu.async_copy(blob.at[pl.ds(OFF_TK, V * D)], tk_v, sem),
        pltpu.async_copy(blob.at[pl.ds(OFF_PB, 16 * 16)], pb_v, sem),
    ]
    for cp in cps:
        cp.wait()

    iota = lax.iota(jnp.int32, 16)
    iota4 = iota * 4
    iota2 = iota * 2
    iotaL = iota * L

    for t, (tv, sv) in enumerate(((tu_v, su_v), (tc_v, sc_v), (tk_v, sk_v))):
        for c in range(V // 16):
            acc = jnp.zeros((16,), jnp.float32)
            for d in range(D):
                col = _f32(plsc.load_gather(tv, [iota4 + (c * 64 + d)]))
                acc = acc + col * _f32(pb_v[pl.ds((t * D + d) * 16, 16)])
            sv[pl.ds(c * 16, 16)] = acc

    w12 = _f32(pb_v[pl.ds(NUM_TABLES * D * 16, 16)])
    w13 = _f32(pb_v[pl.ds((NUM_TABLES * D + 1) * 16, 16)])
    bias = _f32(pb_v[pl.ds((NUM_TABLES * D + 2) * 16, 16)])

    for g in range(GROUPS):
        off = g * (16 * L)
        d0 = _f32(plsc.load_gather(dn_v, [iota2 + g * 32]))
        d1 = _f32(plsc.load_gather(dn_v, [iota2 + g * 32 + 1]))
        acc = bias + d0 * w12 + d1 * w13
        for iv, sv in ((ui_v, su_v), (ci_v, sc_v), (ki_v, sk_v)):
            for l in range(L):
                vi = plsc.load_gather(iv, [iotaL + (off + l)])
                acc = acc + plsc.load_gather(sv, [vi])
        out_v[pl.ds(g * 16, 16)] = acc

    pltpu.sync_copy(out_v, out.at[pl.ds(base, ROWS_PER_W)])


def _bits(x):
    return lax.bitcast_convert_type(x.astype(jnp.float32), jnp.int32).reshape(-1)


@jax.jit
def kernel(user_indices, context_indices, candidate_indices, dense_features,
           table_user, table_context, table_candidate, W, b):
    params = jnp.concatenate([W.astype(jnp.float32).reshape(-1),
                              b.astype(jnp.float32).reshape(-1),
                              jnp.zeros((1,), jnp.float32)])
    pb = jnp.broadcast_to(params[:, None], (16, 16))
    blob = jnp.concatenate([
        user_indices.astype(jnp.int32).reshape(-1),
        context_indices.astype(jnp.int32).reshape(-1),
        candidate_indices.astype(jnp.int32).reshape(-1),
        _bits(dense_features),
        _bits(table_user),
        _bits(table_context),
        _bits(table_candidate),
        _bits(pb),
    ])

    run = functools.partial(
        pl.kernel,
        mesh=plsc.VectorSubcoreMesh(core_axis_name="c", subcore_axis_name="s"),
        out_type=jax.ShapeDtypeStruct((B,), jnp.float32),
        compiler_params=pltpu.CompilerParams(needs_layout_passes=False),
        scratch_types=[
            pltpu.VMEM((ROWS_PER_W * L,), jnp.int32),
            pltpu.VMEM((ROWS_PER_W * L,), jnp.int32),
            pltpu.VMEM((ROWS_PER_W * L,), jnp.int32),
            pltpu.VMEM((ROWS_PER_W * 2,), jnp.int32),
            pltpu.VMEM((V * D,), jnp.int32),
            pltpu.VMEM((V * D,), jnp.int32),
            pltpu.VMEM((V * D,), jnp.int32),
            pltpu.VMEM((16 * 16,), jnp.int32),
            pltpu.VMEM((V,), jnp.float32),
            pltpu.VMEM((V,), jnp.float32),
            pltpu.VMEM((V,), jnp.float32),
            pltpu.VMEM((ROWS_PER_W,), jnp.float32),
            pltpu.SemaphoreType.DMA,
        ],
    )(_sc_kernel_body)
    return run(blob)


# speedup vs baseline: 1.2562x; 1.1955x over previous
import functools

import jax
import jax.numpy as jnp
from jax import lax
from jax.experimental import pallas as pl
from jax.experimental.pallas import tpu as pltpu
from jax.experimental.pallas import tpu_sc as plsc

B = 4096
L = 20
V = 64
D = 4
NUM_TABLES = 3
NW = 32
ROWS_PER_W = B // NW
GROUPS = ROWS_PER_W // 16

BL = B * L
OFF_PI = 0
OFF_DN = BL
OFF_TU = OFF_DN + B * 2
OFF_TC = OFF_TU + V * D
OFF_TK = OFF_TC + V * D
OFF_PB = OFF_TK + V * D
TOTAL = OFF_PB + 16 * 16


def _f32(x):
    return plsc.bitcast(x, jnp.float32)


def _sc_kernel_body(blob, out,
                    pi_v, dn_v, tu_v, tc_v, tk_v, pb_v,
                    su_v, sc_v, sk_v, out_v, sem):
    wid = lax.axis_index("s") * 2 + lax.axis_index("c")
    base = wid * ROWS_PER_W

    cps = [
        pltpu.async_copy(blob.at[pl.ds(OFF_PI + base * L, ROWS_PER_W * L)], pi_v, sem),
        pltpu.async_copy(blob.at[pl.ds(OFF_DN + base * 2, ROWS_PER_W * 2)], dn_v, sem),
        pltpu.async_copy(blob.at[pl.ds(OFF_TU, V * D)], tu_v, sem),
        pltpu.async_copy(blob.at[pl.ds(OFF_TC, V * D)], tc_v, sem),
        pltpu.async_copy(blob.at[pl.ds(OFF_TK, V * D)], tk_v, sem),
        pltpu.async_copy(blob.at[pl.ds(OFF_PB, 16 * 16)], pb_v, sem),
    ]
    for cp in cps:
        cp.wait()

    iota = lax.iota(jnp.int32, 16)
    iota4 = iota * 4
    iota2 = iota * 2
    iotaL = iota * L

    for t, (tv, sv) in enumerate(((tu_v, su_v), (tc_v, sc_v), (tk_v, sk_v))):
        for c in range(V // 16):
            acc = jnp.zeros((16,), jnp.float32)
            for d in range(D):
                col = _f32(plsc.load_gather(tv, [iota4 + (c * 64 + d)]))
                acc = acc + col * _f32(pb_v[pl.ds((t * D + d) * 16, 16)])
            sv[pl.ds(c * 16, 16)] = acc

    w12 = _f32(pb_v[pl.ds(NUM_TABLES * D * 16, 16)])
    w13 = _f32(pb_v[pl.ds((NUM_TABLES * D + 1) * 16, 16)])
    bias = _f32(pb_v[pl.ds((NUM_TABLES * D + 2) * 16, 16)])

    for g in range(GROUPS):
        off = g * (16 * L)
        d0 = _f32(plsc.load_gather(dn_v, [iota2 + g * 32]))
        d1 = _f32(plsc.load_gather(dn_v, [iota2 + g * 32 + 1]))
        acc = bias + d0 * w12 + d1 * w13
        for l in range(L):
            vi = plsc.load_gather(pi_v, [iotaL + (off + l)])
            acc = acc + plsc.load_gather(su_v, [vi & 255])
            acc = acc + plsc.load_gather(sc_v, [(vi >> 8) & 255])
            acc = acc + plsc.load_gather(sk_v, [vi >> 16])
        out_v[pl.ds(g * 16, 16)] = acc

    pltpu.sync_copy(out_v, out.at[pl.ds(base, ROWS_PER_W)])


def _bits(x):
    return lax.bitcast_convert_type(x.astype(jnp.float32), jnp.int32).reshape(-1)


@jax.jit
def kernel(user_indices, context_indices, candidate_indices, dense_features,
           table_user, table_context, table_candidate, W, b):
    params = jnp.concatenate([W.astype(jnp.float32).reshape(-1),
                              b.astype(jnp.float32).reshape(-1),
                              jnp.zeros((1,), jnp.float32)])
    pb = jnp.broadcast_to(params[:, None], (16, 16))
    packed_idx = (user_indices.astype(jnp.int32)
                  | (context_indices.astype(jnp.int32) << 8)
                  | (candidate_indices.astype(jnp.int32) << 16))
    blob = jnp.concatenate([
        packed_idx.reshape(-1),
        _bits(dense_features),
        _bits(table_user),
        _bits(table_context),
        _bits(table_candidate),
        _bits(pb),
    ])

    run = functools.partial(
        pl.kernel,
        mesh=plsc.VectorSubcoreMesh(core_axis_name="c", subcore_axis_name="s"),
        out_type=jax.ShapeDtypeStruct((B,), jnp.float32),
        compiler_params=pltpu.CompilerParams(needs_layout_passes=False),
        scratch_types=[
            pltpu.VMEM((ROWS_PER_W * L,), jnp.int32),
            pltpu.VMEM((ROWS_PER_W * 2,), jnp.int32),
            pltpu.VMEM((V * D,), jnp.int32),
            pltpu.VMEM((V * D,), jnp.int32),
            pltpu.VMEM((V * D,), jnp.int32),
            pltpu.VMEM((16 * 16,), jnp.int32),
            pltpu.VMEM((V,), jnp.float32),
            pltpu.VMEM((V,), jnp.float32),
            pltpu.VMEM((V,), jnp.float32),
            pltpu.VMEM((ROWS_PER_W,), jnp.float32),
            pltpu.SemaphoreType.DMA,
        ],
    )(_sc_kernel_body)
    return run(blob)
